# baseline (speedup 1.0000x reference)
import math

import jax
import jax.numpy as jnp
from jax.experimental import pallas as pl
from jax.experimental.pallas import tpu as pltpu

_NINP = 1024
_NB = 8
_BH = 256
_TOPK = 2
_DK = 64
_DV = 256
_DKC = 32


def _precompute_body(x_ref, wk_ref, wv_ref, kx_ref, v0_ref):
    x = x_ref[...]
    kx = jnp.dot(x, wk_ref[...], preferred_element_type=jnp.float32)
    kx_ref[...] = kx.astype(jnp.bfloat16)
    v0 = jnp.dot(x, wv_ref[...], preferred_element_type=jnp.float32)
    v0_ref[...] = v0.astype(jnp.bfloat16)


def _step_body(kx_ref, v0_ref, h0_ref, w1_ref, wx_ref, wc_ref, bq_ref, bg_ref,
               out_ref, h_scr):
    t = pl.program_id(0)

    @pl.when(t == 0)
    def _init():
        h_scr[...] = h0_ref[...]

    kx = kx_ref[0].astype(jnp.float32)
    v0 = v0_ref[0].astype(jnp.float32)

    h_list = [h_scr[n] for n in range(_NB)]
    s_cols = []
    gh_list = []
    for n in range(_NB):
        hq = jnp.dot(h_list[n].astype(jnp.bfloat16), w1_ref[n],
                     preferred_element_type=jnp.float32)
        q = hq[:, :_DK] + bq_ref[n]
        qb = q.astype(jnp.bfloat16).astype(jnp.float32)
        s_cols.append(jnp.sum(qb * kx, axis=1, keepdims=True))
        gh_list.append(hq[:, _DK:])
    s = jnp.concatenate(s_cols, axis=1) / 8.0

    m = jnp.maximum(s, 0.0)
    e = jnp.exp(s - m)
    att0 = e / (e + jnp.exp(-m))
    attb = att0.astype(jnp.bfloat16).astype(jnp.float32)

    hn_list = []
    for n in range(_NB):
        xin = (attb[:, n:n + 1] * v0).astype(jnp.bfloat16)
        gx = jnp.dot(xin, wx_ref[n],
                     preferred_element_type=jnp.float32) + bg_ref[n]
        gh = gh_list[n]
        r = jax.nn.sigmoid(gx[:, :_BH] + gh[:, :_BH])
        z = jax.nn.sigmoid(gx[:, _BH:2 * _BH] + gh[:, _BH:2 * _BH])
        g = jnp.tanh(gx[:, 2 * _BH:] + r * gh[:, 2 * _BH:])
        hn_list.append((1.0 - z) * g + z * h_list[n])

    qc_list, kc_list, vc_list = [], [], []
    for n in range(_NB):
        c = jnp.dot(hn_list[n].astype(jnp.bfloat16), wc_ref[n],
                    preferred_element_type=jnp.float32)
        qc_list.append(c[:, :_DKC].astype(jnp.bfloat16).astype(jnp.float32))
        kc_list.append(c[:, _DKC:2 * _DKC].astype(jnp.bfloat16).astype(jnp.float32))
        vc_list.append(c[:, 2 * _DKC:])
    qcs = jnp.stack(qc_list)
    kcs = jnp.stack(kc_list)
    vcs = jnp.stack(vc_list)
    logits = (jnp.sum(qcs[:, None] * kcs[None, :], axis=-1)
              / math.sqrt(_DKC))
    lmax = jnp.max(logits, axis=1, keepdims=True)
    le = jnp.exp(logits - lmax)
    ac = le / jnp.sum(le, axis=1, keepdims=True)
    acb = ac.astype(jnp.bfloat16).astype(jnp.float32)
    vcb = vcs.astype(jnp.bfloat16).astype(jnp.float32)

    r1 = att0[:, None, :]
    r2 = att0[:, :, None]
    n_idx = jax.lax.broadcasted_iota(jnp.int32, (1, _NB, _NB), 1)
    m_idx = jax.lax.broadcasted_iota(jnp.int32, (1, _NB, _NB), 2)
    beats = (r1 > r2) | ((r1 == r2) & (m_idx < n_idx))
    rank = jnp.sum(beats.astype(jnp.int32), axis=2)
    maskf = (rank < _TOPK).astype(jnp.float32)

    for n in range(_NB):
        comm = jnp.sum(acb[n][:, :, None] * vcb, axis=0)
        hn2 = hn_list[n] + comm
        mk = maskf[:, n:n + 1]
        hout = mk * hn2 + (1.0 - mk) * h_list[n]
        h_scr[n] = hout
        out_ref[0, :, n * _BH:(n + 1) * _BH] = hout


def kernel(input, hidden, seq_len, Wq_in, bq_in, Wk_in, Wv_in, Wx, Wh, bg,
           Wq_c, Wk_c, Wv_c):
    seq, batch, ninp = input.shape
    rows = seq * batch
    tile = 256
    grid_pre = rows // tile

    xf = input.reshape(rows, ninp).astype(jnp.bfloat16)
    kx_flat, v0_flat = pl.pallas_call(
        _precompute_body,
        grid=(grid_pre,),
        in_specs=[
            pl.BlockSpec((tile, ninp), lambda i: (i, 0)),
            pl.BlockSpec((ninp, _DK), lambda i: (0, 0)),
            pl.BlockSpec((ninp, _DV), lambda i: (0, 0)),
        ],
        out_specs=[
            pl.BlockSpec((tile, _DK), lambda i: (i, 0)),
            pl.BlockSpec((tile, _DV), lambda i: (i, 0)),
        ],
        out_shape=[
            jax.ShapeDtypeStruct((rows, _DK), jnp.bfloat16),
            jax.ShapeDtypeStruct((rows, _DV), jnp.bfloat16),
        ],
    )(xf, Wk_in.astype(jnp.bfloat16), Wv_in.astype(jnp.bfloat16))

    kx_v = kx_flat.reshape(seq, batch, _DK)
    v0_v = v0_flat.reshape(seq, batch, _DV)
    h0 = hidden.reshape(batch, _NB, _BH).transpose(1, 0, 2)
    w1 = jnp.concatenate([Wq_in, Wh], axis=2).astype(jnp.bfloat16)
    wxb = Wx.astype(jnp.bfloat16)
    wc = jnp.concatenate([Wq_c, Wk_c, Wv_c], axis=2).astype(jnp.bfloat16)

    out = pl.pallas_call(
        _step_body,
        grid=(seq,),
        in_specs=[
            pl.BlockSpec((1, batch, _DK), lambda t: (t, 0, 0)),
            pl.BlockSpec((1, batch, _DV), lambda t: (t, 0, 0)),
            pl.BlockSpec((_NB, batch, _BH), lambda t: (0, 0, 0)),
            pl.BlockSpec((_NB, _BH, _DK + 3 * _BH), lambda t: (0, 0, 0)),
            pl.BlockSpec((_NB, _DV, 3 * _BH), lambda t: (0, 0, 0)),
            pl.BlockSpec((_NB, _BH, 2 * _DKC + _BH), lambda t: (0, 0, 0)),
            pl.BlockSpec((_NB, _DK), lambda t: (0, 0)),
            pl.BlockSpec((_NB, 3 * _BH), lambda t: (0, 0)),
        ],
        out_specs=pl.BlockSpec((1, batch, _NB * _BH), lambda t: (t, 0, 0)),
        out_shape=jax.ShapeDtypeStruct((seq, batch, _NB * _BH), jnp.float32),
        scratch_shapes=[pltpu.VMEM((_NB, batch, _BH), jnp.float32)],
        compiler_params=pltpu.CompilerParams(
            dimension_semantics=("arbitrary",),
        ),
    )(kx_v, v0_v, h0, w1, wxb, wc, bq_in, bg)

    return out


# speedup vs baseline: 1.0298x; 1.0298x over previous
import math

import jax
import jax.numpy as jnp
from jax.experimental import pallas as pl
from jax.experimental.pallas import tpu as pltpu

_NINP = 1024
_NB = 8
_BH = 256
_TOPK = 2
_DK = 64
_DV = 256
_DKC = 32


def _precompute_body(x_ref, wk_ref, wv_ref, kx_ref, v0_ref):
    x = x_ref[...]
    kx = jnp.dot(x, wk_ref[...], preferred_element_type=jnp.float32)
    kx_ref[...] = kx.astype(jnp.bfloat16)
    v0 = jnp.dot(x, wv_ref[...], preferred_element_type=jnp.float32)
    v0_ref[...] = v0.astype(jnp.bfloat16)


def _step_body(*refs):
    (kx_ref, v0_ref, h0_ref) = refs[0:3]
    wq_refs = refs[3:3 + _NB]
    wh_refs = refs[3 + _NB:3 + 2 * _NB]
    wx_refs = refs[3 + 2 * _NB:3 + 3 * _NB]
    wqc_refs = refs[3 + 3 * _NB:3 + 4 * _NB]
    wkc_refs = refs[3 + 4 * _NB:3 + 5 * _NB]
    wvc_refs = refs[3 + 5 * _NB:3 + 6 * _NB]
    bq_ref, bg_ref = refs[3 + 6 * _NB:5 + 6 * _NB]
    out_ref = refs[5 + 6 * _NB]
    h_scr = refs[6 + 6 * _NB]

    t = pl.program_id(0)

    @pl.when(t == 0)
    def _init():
        h_scr[...] = h0_ref[...]

    kx = kx_ref[0].astype(jnp.float32)
    v0 = v0_ref[0].astype(jnp.float32)

    h_list = [h_scr[n] for n in range(_NB)]
    hb_list = [h.astype(jnp.bfloat16) for h in h_list]
    s_cols = []
    gh_list = []
    for n in range(_NB):
        q = jnp.dot(hb_list[n], wq_refs[n][...],
                    preferred_element_type=jnp.float32) + bq_ref[n]
        qb = q.astype(jnp.bfloat16).astype(jnp.float32)
        s_cols.append(jnp.sum(qb * kx, axis=1, keepdims=True))
        gh_list.append(jnp.dot(hb_list[n], wh_refs[n][...],
                               preferred_element_type=jnp.float32))
    s = jnp.concatenate(s_cols, axis=1) / 8.0

    m = jnp.maximum(s, 0.0)
    e = jnp.exp(s - m)
    att0 = e / (e + jnp.exp(-m))
    attb = att0.astype(jnp.bfloat16).astype(jnp.float32)

    hn_list = []
    for n in range(_NB):
        xin = (attb[:, n:n + 1] * v0).astype(jnp.bfloat16)
        gx = jnp.dot(xin, wx_refs[n][...],
                     preferred_element_type=jnp.float32) + bg_ref[n]
        gh = gh_list[n]
        r = jax.nn.sigmoid(gx[:, :_BH] + gh[:, :_BH])
        z = jax.nn.sigmoid(gx[:, _BH:2 * _BH] + gh[:, _BH:2 * _BH])
        g = jnp.tanh(gx[:, 2 * _BH:] + r * gh[:, 2 * _BH:])
        hn_list.append((1.0 - z) * g + z * h_list[n])

    qc_list, kc_list, vc_list = [], [], []
    hnb_list = [hn.astype(jnp.bfloat16) for hn in hn_list]
    for n in range(_NB):
        qc = jnp.dot(hnb_list[n], wqc_refs[n][...],
                     preferred_element_type=jnp.float32)
        kc = jnp.dot(hnb_list[n], wkc_refs[n][...],
                     preferred_element_type=jnp.float32)
        vc = jnp.dot(hnb_list[n], wvc_refs[n][...],
                     preferred_element_type=jnp.float32)
        qc_list.append(qc.astype(jnp.bfloat16).astype(jnp.float32))
        kc_list.append(kc.astype(jnp.bfloat16).astype(jnp.float32))
        vc_list.append(vc)
    qcs = jnp.stack(qc_list)
    kcs = jnp.stack(kc_list)
    vcs = jnp.stack(vc_list)
    logits = (jnp.sum(qcs[:, None] * kcs[None, :], axis=-1)
              / math.sqrt(_DKC))
    lmax = jnp.max(logits, axis=1, keepdims=True)
    le = jnp.exp(logits - lmax)
    ac = le / jnp.sum(le, axis=1, keepdims=True)
    acb = ac.astype(jnp.bfloat16).astype(jnp.float32)
    vcb = vcs.astype(jnp.bfloat16).astype(jnp.float32)

    r1 = att0[:, None, :]
    r2 = att0[:, :, None]
    n_idx = jax.lax.broadcasted_iota(jnp.int32, (1, _NB, _NB), 1)
    m_idx = jax.lax.broadcasted_iota(jnp.int32, (1, _NB, _NB), 2)
    beats = (r1 > r2) | ((r1 == r2) & (m_idx < n_idx))
    rank = jnp.sum(beats.astype(jnp.int32), axis=2)
    maskf = (rank < _TOPK).astype(jnp.float32)

    for n in range(_NB):
        comm = jnp.sum(acb[n][:, :, None] * vcb, axis=0)
        hn2 = hn_list[n] + comm
        mk = maskf[:, n:n + 1]
        hout = mk * hn2 + (1.0 - mk) * h_list[n]
        h_scr[n] = hout
        out_ref[0, :, n * _BH:(n + 1) * _BH] = hout


def _full(shape):
    nd = len(shape)
    return pl.BlockSpec(shape, lambda t, _nd=nd: (0,) * _nd)


def kernel(input, hidden, seq_len, Wq_in, bq_in, Wk_in, Wv_in, Wx, Wh, bg,
           Wq_c, Wk_c, Wv_c):
    seq, batch, ninp = input.shape
    rows = seq * batch
    tile = 256
    grid_pre = rows // tile

    xf = input.reshape(rows, ninp).astype(jnp.bfloat16)
    kx_flat, v0_flat = pl.pallas_call(
        _precompute_body,
        grid=(grid_pre,),
        in_specs=[
            pl.BlockSpec((tile, ninp), lambda i: (i, 0)),
            pl.BlockSpec((ninp, _DK), lambda i: (0, 0)),
            pl.BlockSpec((ninp, _DV), lambda i: (0, 0)),
        ],
        out_specs=[
            pl.BlockSpec((tile, _DK), lambda i: (i, 0)),
            pl.BlockSpec((tile, _DV), lambda i: (i, 0)),
        ],
        out_shape=[
            jax.ShapeDtypeStruct((rows, _DK), jnp.bfloat16),
            jax.ShapeDtypeStruct((rows, _DV), jnp.bfloat16),
        ],
    )(xf, Wk_in.astype(jnp.bfloat16), Wv_in.astype(jnp.bfloat16))

    kx_v = kx_flat.reshape(seq, batch, _DK)
    v0_v = v0_flat.reshape(seq, batch, _DV)
    h0 = hidden.reshape(batch, _NB, _BH).transpose(1, 0, 2)

    wq_l = [Wq_in[n].astype(jnp.bfloat16) for n in range(_NB)]
    wh_l = [Wh[n].astype(jnp.bfloat16) for n in range(_NB)]
    wx_l = [Wx[n].astype(jnp.bfloat16) for n in range(_NB)]
    wqc_l = [Wq_c[n].astype(jnp.bfloat16) for n in range(_NB)]
    wkc_l = [Wk_c[n].astype(jnp.bfloat16) for n in range(_NB)]
    wvc_l = [Wv_c[n].astype(jnp.bfloat16) for n in range(_NB)]

    in_specs = [
        pl.BlockSpec((1, batch, _DK), lambda t: (t, 0, 0)),
        pl.BlockSpec((1, batch, _DV), lambda t: (t, 0, 0)),
        _full((_NB, batch, _BH)),
    ]
    in_specs += [_full((_BH, _DK))] * _NB
    in_specs += [_full((_BH, 3 * _BH))] * _NB
    in_specs += [_full((_DV, 3 * _BH))] * _NB
    in_specs += [_full((_BH, _DKC))] * _NB
    in_specs += [_full((_BH, _DKC))] * _NB
    in_specs += [_full((_BH, _BH))] * _NB
    in_specs += [_full((_NB, _DK)), _full((_NB, 3 * _BH))]

    out = pl.pallas_call(
        _step_body,
        grid=(seq,),
        in_specs=in_specs,
        out_specs=pl.BlockSpec((1, batch, _NB * _BH), lambda t: (t, 0, 0)),
        out_shape=jax.ShapeDtypeStruct((seq, batch, _NB * _BH), jnp.float32),
        scratch_shapes=[pltpu.VMEM((_NB, batch, _BH), jnp.float32)],
        compiler_params=pltpu.CompilerParams(
            dimension_semantics=("arbitrary",),
        ),
    )(kx_v, v0_v, h0, *wq_l, *wh_l, *wx_l, *wqc_l, *wkc_l, *wvc_l, bq_in, bg)

    return out
